# baseline (speedup 1.0000x reference)
import functools
import jax
import jax.numpy as jnp
from jax.experimental import pallas as pl
from jax.experimental.pallas import tpu as pltpu

N = 100000
E = 1600000
IN_FEATS = 13
H = 32
CLS_H = 256
NUM_GIN = 5
ROWS = 2000
GRID = N // ROWS

NEG_INF = float("-inf")


def _ln(z, g, b):
    mu = jnp.mean(z, axis=-1, keepdims=True)
    var = jnp.mean((z - mu) ** 2, axis=-1, keepdims=True)
    return (z - mu) * jax.lax.rsqrt(var + 1e-5) * g + b


def _elu(z):
    return jnp.where(z > 0, z, jnp.exp(z) - 1.0)


def _pre_body(x_ref, w1, b1, g1, bb1, w2, b2, g2, bb2, o_ref):
    x = x_ref[...]
    z = _ln(_elu(jnp.dot(x, w1[...], preferred_element_type=jnp.float32, precision=jax.lax.Precision.HIGHEST) + b1[...]),
            g1[...], bb1[...])
    o_ref[...] = _ln(_elu(jnp.dot(z, w2[...], preferred_element_type=jnp.float32, precision=jax.lax.Precision.HIGHEST) + b2[...]),
                     g2[...], bb2[...])


def _mlp_before(x, p):
    full = lambda shape: pl.BlockSpec(shape, lambda i: (0, 0))
    return pl.pallas_call(
        _pre_body,
        grid=(GRID,),
        in_specs=[
            pl.BlockSpec((ROWS, IN_FEATS), lambda i: (i, 0)),
            full((IN_FEATS, 4 * H)), full((1, 4 * H)), full((1, 4 * H)), full((1, 4 * H)),
            full((4 * H, H)), full((1, H)), full((1, H)), full((1, H)),
        ],
        out_specs=pl.BlockSpec((ROWS, H), lambda i: (i, 0)),
        out_shape=jax.ShapeDtypeStruct((N, H), jnp.float32),
    )(x,
      p["mlpb_w1"], p["mlpb_b1"].reshape(1, -1), p["mlpb_ln1_g"].reshape(1, -1), p["mlpb_ln1_b"].reshape(1, -1),
      p["mlpb_w2"], p["mlpb_b2"].reshape(1, -1), p["mlpb_ln2_g"].reshape(1, -1), p["mlpb_ln2_b"].reshape(1, -1))


def _gin_body(h_ref, agg_ref, w, b, g, bb, eps, o_ref):
    agg = agg_ref[...]
    agg = jnp.where(agg == NEG_INF, 0.0, agg)
    t = (1.0 + eps[0, 0]) * h_ref[...] + agg
    z = jnp.dot(t, w[...], preferred_element_type=jnp.float32, precision=jax.lax.Precision.HIGHEST) + b[...]
    o_ref[...] = _ln(z, g[...], bb[...])


def _gin_dense(h, agg, w, b, g, bb, eps):
    fo = w.shape[1]
    full = lambda shape: pl.BlockSpec(shape, lambda i: (0, 0))
    return pl.pallas_call(
        _gin_body,
        grid=(GRID,),
        in_specs=[
            pl.BlockSpec((ROWS, H), lambda i: (i, 0)),
            pl.BlockSpec((ROWS, H), lambda i: (i, 0)),
            full((H, fo)), full((1, fo)), full((1, fo)), full((1, fo)), full((1, 1)),
        ],
        out_specs=pl.BlockSpec((ROWS, fo), lambda i: (i, 0)),
        out_shape=jax.ShapeDtypeStruct((N, fo), jnp.float32),
    )(h, agg, w, b.reshape(1, -1), g.reshape(1, -1), bb.reshape(1, -1), eps.reshape(1, 1))


def _gin_final_body(h_ref, agg_ref, w, b, g, bb, eps, o_ref, sum_ref):
    agg = agg_ref[...]
    agg = jnp.where(agg == NEG_INF, 0.0, agg)
    t = (1.0 + eps[0, 0]) * h_ref[...] + agg
    z = jnp.dot(t, w[...], preferred_element_type=jnp.float32, precision=jax.lax.Precision.HIGHEST) + b[...]
    out = _ln(z, g[...], bb[...])
    o_ref[...] = out

    @pl.when(pl.program_id(0) == 0)
    def _():
        sum_ref[...] = jnp.zeros_like(sum_ref)

    sum_ref[...] += jnp.sum(out, axis=0, keepdims=True)


def _gin_dense_final(h, agg, w, b, g, bb, eps):
    fo = w.shape[1]
    full = lambda shape: pl.BlockSpec(shape, lambda i: (0, 0))
    return pl.pallas_call(
        _gin_final_body,
        grid=(GRID,),
        in_specs=[
            pl.BlockSpec((ROWS, H), lambda i: (i, 0)),
            pl.BlockSpec((ROWS, H), lambda i: (i, 0)),
            full((H, fo)), full((1, fo)), full((1, fo)), full((1, fo)), full((1, 1)),
        ],
        out_specs=[
            pl.BlockSpec((ROWS, fo), lambda i: (i, 0)),
            pl.BlockSpec((1, fo), lambda i: (0, 0)),
        ],
        out_shape=[
            jax.ShapeDtypeStruct((N, fo), jnp.float32),
            jax.ShapeDtypeStruct((1, fo), jnp.float32),
        ],
    )(h, agg, w, b.reshape(1, -1), g.reshape(1, -1), bb.reshape(1, -1), eps.reshape(1, 1))


def _cls_body(s_ref, w1, b1, w2, b2, o_ref):
    pooled = s_ref[...] * (1.0 / N)
    z = _elu(jnp.dot(pooled, w1[...], preferred_element_type=jnp.float32, precision=jax.lax.Precision.HIGHEST) + b1[...])
    o_ref[...] = jnp.dot(z, w2[...], preferred_element_type=jnp.float32, precision=jax.lax.Precision.HIGHEST) + b2[...]


def _classifier(sums, p):
    full = lambda shape: pl.BlockSpec(shape, lambda: (0, 0))
    return pl.pallas_call(
        _cls_body,
        in_specs=[
            full((1, IN_FEATS)),
            full((IN_FEATS, CLS_H)), full((1, CLS_H)),
            full((CLS_H, 1)), full((1, 1)),
        ],
        out_specs=full((1, 1)),
        out_shape=jax.ShapeDtypeStruct((1, 1), jnp.float32),
    )(sums, p["cls1_w"], p["cls1_b"].reshape(1, -1), p["cls2_w"], p["cls2_b"].reshape(1, 1))


def kernel(x, edge_index, edge_weight, params):
    p = params
    src = edge_index[0]
    dst = edge_index[1]

    h = _mlp_before(x, p)

    for i in range(NUM_GIN):
        msgs = h[src] * edge_weight[:, None]
        agg = jax.ops.segment_max(msgs, dst, num_segments=N)

        if i != NUM_GIN - 1:
            h = _gin_dense(h, agg, p["gin_ws"][i], p["gin_bs"][i],
                           p["lm_g"], p["lm_b"], p["gin_eps"][i])
        else:
            h, sums = _gin_dense_final(h, agg, p["gin_ws"][i], p["gin_bs"][i],
                                       p["lm_last_g"], p["lm_last_b"], p["gin_eps"][i])

    pred = _classifier(sums, p)
    return (pred, h)


# speedup vs baseline: 2.9158x; 2.9158x over previous
import functools
import jax
import jax.numpy as jnp
from jax import lax
from jax.experimental import pallas as pl
from jax.experimental.pallas import tpu as pltpu
from jax.experimental.pallas import tpu_sc as plsc

N = 100000
E = 1600000
IN_FEATS = 13
H = 32
CLS_H = 256
NUM_GIN = 5
ROWS = 2000
GRID = N // ROWS

NEG_INF = float("-inf")


def _ln(z, g, b):
    mu = jnp.mean(z, axis=-1, keepdims=True)
    var = jnp.mean((z - mu) ** 2, axis=-1, keepdims=True)
    return (z - mu) * jax.lax.rsqrt(var + 1e-5) * g + b


def _elu(z):
    return jnp.where(z > 0, z, jnp.exp(z) - 1.0)


def _pre_body(x_ref, w1, b1, g1, bb1, w2, b2, g2, bb2, o_ref):
    x = x_ref[...]
    z = _ln(_elu(jnp.dot(x, w1[...], preferred_element_type=jnp.float32, precision=jax.lax.Precision.HIGHEST) + b1[...]),
            g1[...], bb1[...])
    o_ref[...] = _ln(_elu(jnp.dot(z, w2[...], preferred_element_type=jnp.float32, precision=jax.lax.Precision.HIGHEST) + b2[...]),
                     g2[...], bb2[...])


def _mlp_before(x, p):
    full = lambda shape: pl.BlockSpec(shape, lambda i: (0, 0))
    return pl.pallas_call(
        _pre_body,
        grid=(GRID,),
        in_specs=[
            pl.BlockSpec((ROWS, IN_FEATS), lambda i: (i, 0)),
            full((IN_FEATS, 4 * H)), full((1, 4 * H)), full((1, 4 * H)), full((1, 4 * H)),
            full((4 * H, H)), full((1, H)), full((1, H)), full((1, H)),
        ],
        out_specs=pl.BlockSpec((ROWS, H), lambda i: (i, 0)),
        out_shape=jax.ShapeDtypeStruct((N, H), jnp.float32),
    )(x,
      p["mlpb_w1"], p["mlpb_b1"].reshape(1, -1), p["mlpb_ln1_g"].reshape(1, -1), p["mlpb_ln1_b"].reshape(1, -1),
      p["mlpb_w2"], p["mlpb_b2"].reshape(1, -1), p["mlpb_ln2_g"].reshape(1, -1), p["mlpb_ln2_b"].reshape(1, -1))


def _gin_body(h_ref, agg_ref, w, b, g, bb, eps, o_ref):
    agg = agg_ref[...]
    agg = jnp.where(agg == NEG_INF, 0.0, agg)
    t = (1.0 + eps[0, 0]) * h_ref[...] + agg
    z = jnp.dot(t, w[...], preferred_element_type=jnp.float32, precision=jax.lax.Precision.HIGHEST) + b[...]
    o_ref[...] = _ln(z, g[...], bb[...])


def _gin_dense(h, agg, w, b, g, bb, eps):
    fo = w.shape[1]
    full = lambda shape: pl.BlockSpec(shape, lambda i: (0, 0))
    return pl.pallas_call(
        _gin_body,
        grid=(GRID,),
        in_specs=[
            pl.BlockSpec((ROWS, H), lambda i: (i, 0)),
            pl.BlockSpec((ROWS, H), lambda i: (i, 0)),
            full((H, fo)), full((1, fo)), full((1, fo)), full((1, fo)), full((1, 1)),
        ],
        out_specs=pl.BlockSpec((ROWS, fo), lambda i: (i, 0)),
        out_shape=jax.ShapeDtypeStruct((N, fo), jnp.float32),
    )(h, agg, w, b.reshape(1, -1), g.reshape(1, -1), bb.reshape(1, -1), eps.reshape(1, 1))


def _gin_final_body(h_ref, agg_ref, w, b, g, bb, eps, o_ref, sum_ref):
    agg = agg_ref[...]
    agg = jnp.where(agg == NEG_INF, 0.0, agg)
    t = (1.0 + eps[0, 0]) * h_ref[...] + agg
    z = jnp.dot(t, w[...], preferred_element_type=jnp.float32, precision=jax.lax.Precision.HIGHEST) + b[...]
    out = _ln(z, g[...], bb[...])
    o_ref[...] = out

    @pl.when(pl.program_id(0) == 0)
    def _():
        sum_ref[...] = jnp.zeros_like(sum_ref)

    sum_ref[...] += jnp.sum(out, axis=0, keepdims=True)


def _gin_dense_final(h, agg, w, b, g, bb, eps):
    fo = w.shape[1]
    full = lambda shape: pl.BlockSpec(shape, lambda i: (0, 0))
    return pl.pallas_call(
        _gin_final_body,
        grid=(GRID,),
        in_specs=[
            pl.BlockSpec((ROWS, H), lambda i: (i, 0)),
            pl.BlockSpec((ROWS, H), lambda i: (i, 0)),
            full((H, fo)), full((1, fo)), full((1, fo)), full((1, fo)), full((1, 1)),
        ],
        out_specs=[
            pl.BlockSpec((ROWS, fo), lambda i: (i, 0)),
            pl.BlockSpec((1, fo), lambda i: (0, 0)),
        ],
        out_shape=[
            jax.ShapeDtypeStruct((N, fo), jnp.float32),
            jax.ShapeDtypeStruct((1, fo), jnp.float32),
        ],
    )(h, agg, w, b.reshape(1, -1), g.reshape(1, -1), bb.reshape(1, -1), eps.reshape(1, 1))


def _cls_body(s_ref, w1, b1, w2, b2, o_ref):
    pooled = s_ref[...] * (1.0 / N)
    z = _elu(jnp.dot(pooled, w1[...], preferred_element_type=jnp.float32, precision=jax.lax.Precision.HIGHEST) + b1[...])
    o_ref[...] = jnp.dot(z, w2[...], preferred_element_type=jnp.float32, precision=jax.lax.Precision.HIGHEST) + b2[...]


def _classifier(sums, p):
    full = lambda shape: pl.BlockSpec(shape, lambda: (0, 0))
    return pl.pallas_call(
        _cls_body,
        in_specs=[
            full((1, IN_FEATS)),
            full((IN_FEATS, CLS_H)), full((1, CLS_H)),
            full((CLS_H, 1)), full((1, 1)),
        ],
        out_specs=full((1, 1)),
        out_shape=jax.ShapeDtypeStruct((1, 1), jnp.float32),
    )(sums, p["cls1_w"], p["cls1_b"].reshape(1, -1), p["cls2_w"], p["cls2_b"].reshape(1, 1))


NW = 32
NC = 2
SLICE_E = E // NW
BUCKET_N = N // NW
CHUNK_A = 2000
Q = 256
CAP = ((SLICE_E + Q - 1) // Q + 2) * Q
DUMMY_ROW = BUCKET_N
SENT = BUCKET_N + 1


def _worker_id():
    return lax.axis_index("s") * NC + lax.axis_index("c")


def _partition_body(src_hbm, dst_hbm, w_hbm, rec_src, rec_dl, rec_w,
                    srcc, dstc, wc, bvv, dlvv, st_src, st_dl, st_w, cnt_s):
    wid = _worker_id()

    def clr(b, _):
        cnt_s[b] = 0
        return 0
    lax.fori_loop(0, NW, clr, 0)

    def chunk_body(c, _):
        base = pl.multiple_of(wid * SLICE_E + c * CHUNK_A, 8)
        pltpu.sync_copy(src_hbm.at[pl.ds(base, CHUNK_A)], srcc)
        pltpu.sync_copy(dst_hbm.at[pl.ds(base, CHUNK_A)], dstc)
        pltpu.sync_copy(w_hbm.at[pl.ds(base, CHUNK_A)], wc)

        def vec_body(j, _):
            d = dstc[pl.ds(j * 16, 16)]
            b0 = lax.shift_right_logical(d * 10738, 25)
            b = jnp.where(d < b0 * BUCKET_N, b0 - 1, b0)
            bvv[pl.ds(j * 16, 16)] = b
            dlvv[pl.ds(j * 16, 16)] = d - b * BUCKET_N
            return 0
        lax.fori_loop(0, CHUNK_A // 16, vec_body, 0)

        def bucket_body(b, _):
            sb = b * (Q + 16)

            def grp(j, pos):
                bb = bvv[pl.ds(j * 16, 16)]
                m = bb == b
                o = pos & (Q - 1)
                plsc.store_compressed(st_src.at[pl.ds(sb + o, 16)], srcc[pl.ds(j * 16, 16)], mask=m)
                plsc.store_compressed(st_dl.at[pl.ds(sb + o, 16)], dlvv[pl.ds(j * 16, 16)], mask=m)
                plsc.store_compressed(st_w.at[pl.ds(sb + o, 16)], wc[pl.ds(j * 16, 16)], mask=m)
                pcv = plsc.all_reduce_population_count(m)
                pos2 = pos + pcv[0]

                @pl.when(lax.shift_right_logical(pos2, 8) > lax.shift_right_logical(pos, 8))
                def _():
                    k = lax.shift_right_logical(pos, 8)
                    pltpu.sync_copy(st_src.at[pl.ds(sb, Q)], rec_src.at[pl.ds(pl.multiple_of((wid * NW + b) * CAP + k * Q, Q), Q)])
                    pltpu.sync_copy(st_dl.at[pl.ds(sb, Q)], rec_dl.at[pl.ds(pl.multiple_of((wid * NW + b) * CAP + k * Q, Q), Q)])
                    pltpu.sync_copy(st_w.at[pl.ds(sb, Q)], rec_w.at[pl.ds(pl.multiple_of((wid * NW + b) * CAP + k * Q, Q), Q)])
                    st_src[pl.ds(sb, 16)] = st_src[pl.ds(sb + Q, 16)]
                    st_dl[pl.ds(sb, 16)] = st_dl[pl.ds(sb + Q, 16)]
                    st_w[pl.ds(sb, 16)] = st_w[pl.ds(sb + Q, 16)]
                return pos2
            cnt_s[b] = lax.fori_loop(0, CHUNK_A // 16, grp, cnt_s[b])
            return 0
        lax.fori_loop(0, NW, bucket_body, 0)
        return 0
    lax.fori_loop(0, SLICE_E // CHUNK_A, chunk_body, 0)

    lanes = lax.iota(jnp.int32, 16)
    zi = jnp.zeros((16,), jnp.int32)
    zf = jnp.zeros((16,), jnp.float32)
    dumv = jnp.full((16,), DUMMY_ROW, jnp.int32)
    sentv = jnp.full((16,), SENT, jnp.int32)

    def tail_body(b, _):
        sb = b * (Q + 16)
        pos = cnt_s[b]
        o = pos & (Q - 1)

        @pl.when(o != 0)
        def _():
            def padw(wnd, _):
                ws = wnd * 16
                keep = (ws + lanes) < o
                st_src[pl.ds(sb + ws, 16)] = jnp.where(keep, st_src[pl.ds(sb + ws, 16)], zi)
                st_dl[pl.ds(sb + ws, 16)] = jnp.where(keep, st_dl[pl.ds(sb + ws, 16)], dumv)
                st_w[pl.ds(sb + ws, 16)] = jnp.where(keep, st_w[pl.ds(sb + ws, 16)], zf)
                return 0
            lax.fori_loop(0, Q // 16, padw, 0)
            k = lax.shift_right_logical(pos, 8)
            pltpu.sync_copy(st_src.at[pl.ds(sb, Q)], rec_src.at[pl.ds(pl.multiple_of((wid * NW + b) * CAP + k * Q, Q), Q)])
            pltpu.sync_copy(st_dl.at[pl.ds(sb, Q)], rec_dl.at[pl.ds(pl.multiple_of((wid * NW + b) * CAP + k * Q, Q), Q)])
            pltpu.sync_copy(st_w.at[pl.ds(sb, Q)], rec_w.at[pl.ds(pl.multiple_of((wid * NW + b) * CAP + k * Q, Q), Q)])

        pc = (pos + Q - 1) & (-Q)

        def sentw(wnd, _):
            ws = wnd * 16
            st_src[pl.ds(sb + ws, 16)] = zi
            st_dl[pl.ds(sb + ws, 16)] = sentv
            st_w[pl.ds(sb + ws, 16)] = zf
            return 0
        lax.fori_loop(0, Q // 16, sentw, 0)
        pltpu.sync_copy(st_src.at[pl.ds(sb, Q)], rec_src.at[pl.ds(pl.multiple_of((wid * NW + b) * CAP + pc, Q), Q)])
        pltpu.sync_copy(st_dl.at[pl.ds(sb, Q)], rec_dl.at[pl.ds(pl.multiple_of((wid * NW + b) * CAP + pc, Q), Q)])
        pltpu.sync_copy(st_w.at[pl.ds(sb, Q)], rec_w.at[pl.ds(pl.multiple_of((wid * NW + b) * CAP + pc, Q), Q)])
        return 0
    lax.fori_loop(0, NW, tail_body, 0)


def _sc_partition(src, dst, w):
    mesh = plsc.VectorSubcoreMesh(core_axis_name="c", subcore_axis_name="s")
    rec_shape = (NW * NW * CAP,)
    return pl.kernel(
        _partition_body,
        out_type=[
            jax.ShapeDtypeStruct(rec_shape, jnp.int32),
            jax.ShapeDtypeStruct(rec_shape, jnp.int32),
            jax.ShapeDtypeStruct(rec_shape, jnp.float32),
        ],
        mesh=mesh,
        compiler_params=pltpu.CompilerParams(needs_layout_passes=False, use_tc_tiling_on_sc=False),
        scratch_types=[
            pltpu.VMEM((CHUNK_A,), jnp.int32),
            pltpu.VMEM((CHUNK_A,), jnp.int32),
            pltpu.VMEM((CHUNK_A,), jnp.float32),
            pltpu.VMEM((CHUNK_A,), jnp.int32),
            pltpu.VMEM((CHUNK_A,), jnp.int32),
            pltpu.VMEM((NW * (Q + 16),), jnp.int32),
            pltpu.VMEM((NW * (Q + 16),), jnp.int32),
            pltpu.VMEM((NW * (Q + 16),), jnp.float32),
            pltpu.SMEM((NW,), jnp.int32),
        ],
    )(src, dst, w)


def _scatter_body(h_hbm, rec_src, rec_dl, rec_w, agg_hbm,
                  acc, msg, srcb, dlb, wb, gsem):
    t = _worker_id()
    neg = jnp.full((16,), NEG_INF, jnp.float32)

    def initb(i, _):
        acc[pl.ds(i * 16, 16)] = neg
        return 0
    lax.fori_loop(0, (BUCKET_N + 1) * H // 16, initb, 0)

    def frag(s, _):
        def cond(carry):
            return carry[1]

        def wbody(carry):
            c, _ = carry
            buf = c & 1
            pltpu.sync_copy(rec_dl.at[pl.ds(pl.multiple_of((s * NW + t) * CAP + c * Q, Q), Q)], dlb.at[buf])
            pltpu.sync_copy(rec_src.at[pl.ds(pl.multiple_of((s * NW + t) * CAP + c * Q, Q), Q)], srcb.at[buf])
            pltpu.sync_copy(rec_w.at[pl.ds(pl.multiple_of((s * NW + t) * CAP + c * Q, Q), Q)], wb.at[buf])
            v0 = dlb[buf, pl.ds(0, 16)]
            go = v0[0] < SENT

            @pl.when(go)
            def _():
                d0 = pltpu.async_copy(h_hbm.at[srcb.at[buf, pl.ds(0, 128)]],
                                      msg.at[buf, pl.ds(0, 128)], gsem)
                d1 = pltpu.async_copy(h_hbm.at[srcb.at[buf, pl.ds(128, 128)]],
                                      msg.at[buf, pl.ds(128, 128)], gsem)
                d0.wait()
                d1.wait()

                def grp(gi, _):
                    base = gi * 16
                    dlv = dlb[buf, pl.ds(base, 16)]
                    wv16 = wb[buf, pl.ds(base, 16)]
                    for j in range(16):
                        a = dlv[j] * H
                        wv = wv16[j]
                        e = base + j
                        m0 = msg[buf, e, pl.ds(0, 16)] * wv
                        m1 = msg[buf, e, pl.ds(16, 16)] * wv
                        acc[pl.ds(a, 16)] = jnp.maximum(acc[pl.ds(a, 16)], m0)
                        acc[pl.ds(a + 16, 16)] = jnp.maximum(acc[pl.ds(a + 16, 16)], m1)
                    return 0
                lax.fori_loop(0, Q // 16, grp, 0)
            return (c + 1, go)
        lax.while_loop(cond, wbody, (0, True))
        return 0
    lax.fori_loop(0, NW, frag, 0)

    pltpu.sync_copy(acc.at[pl.ds(0, BUCKET_N * H)],
                    agg_hbm.at[pl.ds(pl.multiple_of(t * BUCKET_N * H, 32), BUCKET_N * H)])


def _sc_segment_max(h, rec_src, rec_dl, rec_w):
    mesh = plsc.VectorSubcoreMesh(core_axis_name="c", subcore_axis_name="s")
    return pl.kernel(
        _scatter_body,
        out_type=jax.ShapeDtypeStruct((N * H,), jnp.float32),
        mesh=mesh,
        compiler_params=pltpu.CompilerParams(needs_layout_passes=False, use_tc_tiling_on_sc=False),
        scratch_types=[
            pltpu.VMEM(((BUCKET_N + 1) * H,), jnp.float32),
            pltpu.VMEM((2, Q, H), jnp.float32),
            pltpu.VMEM((2, Q), jnp.int32),
            pltpu.VMEM((2, Q), jnp.int32),
            pltpu.VMEM((2, Q), jnp.float32),
            pltpu.SemaphoreType.DMA,
        ],
    )(h, rec_src, rec_dl, rec_w)


def kernel(x, edge_index, edge_weight, params):
    p = params
    src = edge_index[0]
    dst = edge_index[1]

    h = _mlp_before(x, p)
    rec_src, rec_dl, rec_w = _sc_partition(src, dst, edge_weight)

    for i in range(NUM_GIN):
        agg = _sc_segment_max(h, rec_src, rec_dl, rec_w).reshape(N, H)

        if i != NUM_GIN - 1:
            h = _gin_dense(h, agg, p["gin_ws"][i], p["gin_bs"][i],
                           p["lm_g"], p["lm_b"], p["gin_eps"][i])
        else:
            h, sums = _gin_dense_final(h, agg, p["gin_ws"][i], p["gin_bs"][i],
                                       p["lm_last_g"], p["lm_last_b"], p["gin_eps"][i])

    pred = _classifier(sums, p)
    return (pred, h)


# speedup vs baseline: 2.9901x; 1.0255x over previous
import functools
import jax
import jax.numpy as jnp
from jax import lax
from jax.experimental import pallas as pl
from jax.experimental.pallas import tpu as pltpu
from jax.experimental.pallas import tpu_sc as plsc

N = 100000
E = 1600000
IN_FEATS = 13
H = 32
CLS_H = 256
NUM_GIN = 5
ROWS = 2000
GRID = N // ROWS

NEG_INF = float("-inf")


def _ln(z, g, b):
    mu = jnp.mean(z, axis=-1, keepdims=True)
    var = jnp.mean((z - mu) ** 2, axis=-1, keepdims=True)
    return (z - mu) * jax.lax.rsqrt(var + 1e-5) * g + b


def _elu(z):
    return jnp.where(z > 0, z, jnp.exp(z) - 1.0)


def _pre_body(x_ref, w1, b1, g1, bb1, w2, b2, g2, bb2, o_ref):
    x = x_ref[...]
    z = _ln(_elu(jnp.dot(x, w1[...], preferred_element_type=jnp.float32, precision=jax.lax.Precision.HIGHEST) + b1[...]),
            g1[...], bb1[...])
    o_ref[...] = _ln(_elu(jnp.dot(z, w2[...], preferred_element_type=jnp.float32, precision=jax.lax.Precision.HIGHEST) + b2[...]),
                     g2[...], bb2[...])


def _mlp_before(x, p):
    full = lambda shape: pl.BlockSpec(shape, lambda i: (0, 0))
    return pl.pallas_call(
        _pre_body,
        grid=(GRID,),
        in_specs=[
            pl.BlockSpec((ROWS, IN_FEATS), lambda i: (i, 0)),
            full((IN_FEATS, 4 * H)), full((1, 4 * H)), full((1, 4 * H)), full((1, 4 * H)),
            full((4 * H, H)), full((1, H)), full((1, H)), full((1, H)),
        ],
        out_specs=pl.BlockSpec((ROWS, H), lambda i: (i, 0)),
        out_shape=jax.ShapeDtypeStruct((N, H), jnp.float32),
    )(x,
      p["mlpb_w1"], p["mlpb_b1"].reshape(1, -1), p["mlpb_ln1_g"].reshape(1, -1), p["mlpb_ln1_b"].reshape(1, -1),
      p["mlpb_w2"], p["mlpb_b2"].reshape(1, -1), p["mlpb_ln2_g"].reshape(1, -1), p["mlpb_ln2_b"].reshape(1, -1))


def _gin_body(h_ref, agg_ref, w, b, g, bb, eps, o_ref):
    agg = agg_ref[...]
    agg = jnp.where(agg == NEG_INF, 0.0, agg)
    t = (1.0 + eps[0, 0]) * h_ref[...] + agg
    z = jnp.dot(t, w[...], preferred_element_type=jnp.float32, precision=jax.lax.Precision.HIGHEST) + b[...]
    o_ref[...] = _ln(z, g[...], bb[...])


def _gin_dense(h, agg, w, b, g, bb, eps):
    fo = w.shape[1]
    full = lambda shape: pl.BlockSpec(shape, lambda i: (0, 0))
    return pl.pallas_call(
        _gin_body,
        grid=(GRID,),
        in_specs=[
            pl.BlockSpec((ROWS, H), lambda i: (i, 0)),
            pl.BlockSpec((ROWS, H), lambda i: (i, 0)),
            full((H, fo)), full((1, fo)), full((1, fo)), full((1, fo)), full((1, 1)),
        ],
        out_specs=pl.BlockSpec((ROWS, fo), lambda i: (i, 0)),
        out_shape=jax.ShapeDtypeStruct((N, fo), jnp.float32),
    )(h, agg, w, b.reshape(1, -1), g.reshape(1, -1), bb.reshape(1, -1), eps.reshape(1, 1))


def _gin_final_body(h_ref, agg_ref, w, b, g, bb, eps, o_ref, sum_ref):
    agg = agg_ref[...]
    agg = jnp.where(agg == NEG_INF, 0.0, agg)
    t = (1.0 + eps[0, 0]) * h_ref[...] + agg
    z = jnp.dot(t, w[...], preferred_element_type=jnp.float32, precision=jax.lax.Precision.HIGHEST) + b[...]
    out = _ln(z, g[...], bb[...])
    o_ref[...] = out

    @pl.when(pl.program_id(0) == 0)
    def _():
        sum_ref[...] = jnp.zeros_like(sum_ref)

    sum_ref[...] += jnp.sum(out, axis=0, keepdims=True)


def _gin_dense_final(h, agg, w, b, g, bb, eps):
    fo = w.shape[1]
    full = lambda shape: pl.BlockSpec(shape, lambda i: (0, 0))
    return pl.pallas_call(
        _gin_final_body,
        grid=(GRID,),
        in_specs=[
            pl.BlockSpec((ROWS, H), lambda i: (i, 0)),
            pl.BlockSpec((ROWS, H), lambda i: (i, 0)),
            full((H, fo)), full((1, fo)), full((1, fo)), full((1, fo)), full((1, 1)),
        ],
        out_specs=[
            pl.BlockSpec((ROWS, fo), lambda i: (i, 0)),
            pl.BlockSpec((1, fo), lambda i: (0, 0)),
        ],
        out_shape=[
            jax.ShapeDtypeStruct((N, fo), jnp.float32),
            jax.ShapeDtypeStruct((1, fo), jnp.float32),
        ],
    )(h, agg, w, b.reshape(1, -1), g.reshape(1, -1), bb.reshape(1, -1), eps.reshape(1, 1))


def _cls_body(s_ref, w1, b1, w2, b2, o_ref):
    pooled = s_ref[...] * (1.0 / N)
    z = _elu(jnp.dot(pooled, w1[...], preferred_element_type=jnp.float32, precision=jax.lax.Precision.HIGHEST) + b1[...])
    o_ref[...] = jnp.dot(z, w2[...], preferred_element_type=jnp.float32, precision=jax.lax.Precision.HIGHEST) + b2[...]


def _classifier(sums, p):
    full = lambda shape: pl.BlockSpec(shape, lambda: (0, 0))
    return pl.pallas_call(
        _cls_body,
        in_specs=[
            full((1, IN_FEATS)),
            full((IN_FEATS, CLS_H)), full((1, CLS_H)),
            full((CLS_H, 1)), full((1, 1)),
        ],
        out_specs=full((1, 1)),
        out_shape=jax.ShapeDtypeStruct((1, 1), jnp.float32),
    )(sums, p["cls1_w"], p["cls1_b"].reshape(1, -1), p["cls2_w"], p["cls2_b"].reshape(1, 1))


NW = 32
NC = 2
SLICE_E = E // NW
BUCKET_N = N // NW
CHUNK_A = 2000
Q = 256
CAP = ((SLICE_E + Q - 1) // Q + 2) * Q
DUMMY_ROW = BUCKET_N
SENT = BUCKET_N + 1


def _worker_id():
    return lax.axis_index("s") * NC + lax.axis_index("c")


def _partition_body(src_hbm, dst_hbm, w_hbm, rec_src, rec_dl, rec_w,
                    srcc, dstc, wc, bvv, dlvv, st_src, st_dl, st_w, cnt_s):
    wid = _worker_id()

    def clr(b, _):
        cnt_s[b] = 0
        return 0
    lax.fori_loop(0, NW, clr, 0)

    def chunk_body(c, _):
        base = pl.multiple_of(wid * SLICE_E + c * CHUNK_A, 8)
        pltpu.sync_copy(src_hbm.at[pl.ds(base, CHUNK_A)], srcc)
        pltpu.sync_copy(dst_hbm.at[pl.ds(base, CHUNK_A)], dstc)
        pltpu.sync_copy(w_hbm.at[pl.ds(base, CHUNK_A)], wc)

        def vec_body(j, _):
            d = dstc[pl.ds(j * 16, 16)]
            b0 = lax.shift_right_logical(d * 10738, 25)
            b = jnp.where(d < b0 * BUCKET_N, b0 - 1, b0)
            bvv[pl.ds(j * 16, 16)] = b
            dlvv[pl.ds(j * 16, 16)] = d - b * BUCKET_N
            return 0
        lax.fori_loop(0, CHUNK_A // 16, vec_body, 0)

        def bucket_body(b, _):
            sb = b * (Q + 16)

            def grp(j, pos):
                bb = bvv[pl.ds(j * 16, 16)]
                m = bb == b
                o = pos & (Q - 1)
                plsc.store_compressed(st_src.at[pl.ds(sb + o, 16)], srcc[pl.ds(j * 16, 16)], mask=m)
                plsc.store_compressed(st_dl.at[pl.ds(sb + o, 16)], dlvv[pl.ds(j * 16, 16)], mask=m)
                plsc.store_compressed(st_w.at[pl.ds(sb + o, 16)], wc[pl.ds(j * 16, 16)], mask=m)
                pcv = plsc.all_reduce_population_count(m)
                pos2 = pos + pcv[0]

                @pl.when(lax.shift_right_logical(pos2, 8) > lax.shift_right_logical(pos, 8))
                def _():
                    k = lax.shift_right_logical(pos, 8)
                    pltpu.sync_copy(st_src.at[pl.ds(sb, Q)], rec_src.at[pl.ds(pl.multiple_of((wid * NW + b) * CAP + k * Q, Q), Q)])
                    pltpu.sync_copy(st_dl.at[pl.ds(sb, Q)], rec_dl.at[pl.ds(pl.multiple_of((wid * NW + b) * CAP + k * Q, Q), Q)])
                    pltpu.sync_copy(st_w.at[pl.ds(sb, Q)], rec_w.at[pl.ds(pl.multiple_of((wid * NW + b) * CAP + k * Q, Q), Q)])
                    st_src[pl.ds(sb, 16)] = st_src[pl.ds(sb + Q, 16)]
                    st_dl[pl.ds(sb, 16)] = st_dl[pl.ds(sb + Q, 16)]
                    st_w[pl.ds(sb, 16)] = st_w[pl.ds(sb + Q, 16)]
                return pos2
            cnt_s[b] = lax.fori_loop(0, CHUNK_A // 16, grp, cnt_s[b])
            return 0
        lax.fori_loop(0, NW, bucket_body, 0)
        return 0
    lax.fori_loop(0, SLICE_E // CHUNK_A, chunk_body, 0)

    lanes = lax.iota(jnp.int32, 16)
    zi = jnp.zeros((16,), jnp.int32)
    zf = jnp.zeros((16,), jnp.float32)
    dumv = jnp.full((16,), DUMMY_ROW, jnp.int32)
    sentv = jnp.full((16,), SENT, jnp.int32)

    def tail_body(b, _):
        sb = b * (Q + 16)
        pos = cnt_s[b]
        o = pos & (Q - 1)

        @pl.when(o != 0)
        def _():
            def padw(wnd, _):
                ws = wnd * 16
                keep = (ws + lanes) < o
                st_src[pl.ds(sb + ws, 16)] = jnp.where(keep, st_src[pl.ds(sb + ws, 16)], zi)
                st_dl[pl.ds(sb + ws, 16)] = jnp.where(keep, st_dl[pl.ds(sb + ws, 16)], dumv)
                st_w[pl.ds(sb + ws, 16)] = jnp.where(keep, st_w[pl.ds(sb + ws, 16)], zf)
                return 0
            lax.fori_loop(0, Q // 16, padw, 0)
            k = lax.shift_right_logical(pos, 8)
            pltpu.sync_copy(st_src.at[pl.ds(sb, Q)], rec_src.at[pl.ds(pl.multiple_of((wid * NW + b) * CAP + k * Q, Q), Q)])
            pltpu.sync_copy(st_dl.at[pl.ds(sb, Q)], rec_dl.at[pl.ds(pl.multiple_of((wid * NW + b) * CAP + k * Q, Q), Q)])
            pltpu.sync_copy(st_w.at[pl.ds(sb, Q)], rec_w.at[pl.ds(pl.multiple_of((wid * NW + b) * CAP + k * Q, Q), Q)])

        pc = (pos + Q - 1) & (-Q)

        def sentw(wnd, _):
            ws = wnd * 16
            st_src[pl.ds(sb + ws, 16)] = zi
            st_dl[pl.ds(sb + ws, 16)] = sentv
            st_w[pl.ds(sb + ws, 16)] = zf
            return 0
        lax.fori_loop(0, Q // 16, sentw, 0)
        pltpu.sync_copy(st_src.at[pl.ds(sb, Q)], rec_src.at[pl.ds(pl.multiple_of((wid * NW + b) * CAP + pc, Q), Q)])
        pltpu.sync_copy(st_dl.at[pl.ds(sb, Q)], rec_dl.at[pl.ds(pl.multiple_of((wid * NW + b) * CAP + pc, Q), Q)])
        pltpu.sync_copy(st_w.at[pl.ds(sb, Q)], rec_w.at[pl.ds(pl.multiple_of((wid * NW + b) * CAP + pc, Q), Q)])
        return 0
    lax.fori_loop(0, NW, tail_body, 0)


def _sc_partition(src, dst, w):
    mesh = plsc.VectorSubcoreMesh(core_axis_name="c", subcore_axis_name="s")
    rec_shape = (NW * NW * CAP,)
    return pl.kernel(
        _partition_body,
        out_type=[
            jax.ShapeDtypeStruct(rec_shape, jnp.int32),
            jax.ShapeDtypeStruct(rec_shape, jnp.int32),
            jax.ShapeDtypeStruct(rec_shape, jnp.float32),
        ],
        mesh=mesh,
        compiler_params=pltpu.CompilerParams(needs_layout_passes=False, use_tc_tiling_on_sc=False),
        scratch_types=[
            pltpu.VMEM((CHUNK_A,), jnp.int32),
            pltpu.VMEM((CHUNK_A,), jnp.int32),
            pltpu.VMEM((CHUNK_A,), jnp.float32),
            pltpu.VMEM((CHUNK_A,), jnp.int32),
            pltpu.VMEM((CHUNK_A,), jnp.int32),
            pltpu.VMEM((NW * (Q + 16),), jnp.int32),
            pltpu.VMEM((NW * (Q + 16),), jnp.int32),
            pltpu.VMEM((NW * (Q + 16),), jnp.float32),
            pltpu.SMEM((NW,), jnp.int32),
        ],
    )(src, dst, w)


def _scatter_body(h_hbm, rec_src, rec_dl, rec_w, agg_hbm,
                  acc, msg, srcb, dlb, wb, msem, gsem):
    t = _worker_id()
    neg = jnp.full((16,), NEG_INF, jnp.float32)

    def initb(i, _):
        acc[pl.ds(i * 16, 16)] = neg
        return 0
    lax.fori_loop(0, (BUCKET_N + 1) * H // 16, initb, 0)

    def frag(s, _):
        fbase = (s * NW + t) * CAP

        def meta_desc(c):
            slot = c % 3
            off = pl.multiple_of(fbase + c * Q, Q)
            return (pltpu.make_async_copy(rec_dl.at[pl.ds(off, Q)], dlb.at[slot], msem.at[slot]),
                    pltpu.make_async_copy(rec_src.at[pl.ds(off, Q)], srcb.at[slot], msem.at[slot]),
                    pltpu.make_async_copy(rec_w.at[pl.ds(off, Q)], wb.at[slot], msem.at[slot]))

        def meta_issue(c):
            for d in meta_desc(c):
                d.start()

        def meta_wait(c):
            for d in meta_desc(c):
                d.wait()

        def gather_desc(c):
            mslot = c % 3
            gslot = c & 1
            return (pltpu.make_async_copy(h_hbm.at[srcb.at[mslot, pl.ds(0, 128)]],
                                          msg.at[gslot, pl.ds(0, 128)], gsem.at[gslot]),
                    pltpu.make_async_copy(h_hbm.at[srcb.at[mslot, pl.ds(128, 128)]],
                                          msg.at[gslot, pl.ds(128, 128)], gsem.at[gslot]))

        def gather_issue(c):
            for d in gather_desc(c):
                d.start()

        def gather_wait(c):
            for d in gather_desc(c):
                d.wait()

        def check_go(c):
            v0 = dlb[c % 3, pl.ds(0, 16)]
            return v0[0] < SENT

        meta_issue(0)
        meta_issue(1)
        meta_wait(0)
        go0 = check_go(0)

        @pl.when(go0)
        def _():
            gather_issue(0)

        def cond(carry):
            return carry[1]

        def wbody(carry):
            c, _ = carry
            buf = c & 1
            mslot = c % 3
            meta_issue(c + 2)
            meta_wait(c + 1)
            go1 = check_go(c + 1)

            @pl.when(go1)
            def _():
                gather_issue(c + 1)
            gather_wait(c)

            def grp(gi, _):
                base = gi * 16
                dlv = dlb[mslot, pl.ds(base, 16)]
                wv16 = wb[mslot, pl.ds(base, 16)]
                for j in range(16):
                    a = dlv[j] * H
                    wv = wv16[j]
                    e = base + j
                    m0 = msg[buf, e, pl.ds(0, 16)] * wv
                    m1 = msg[buf, e, pl.ds(16, 16)] * wv
                    acc[pl.ds(a, 16)] = jnp.maximum(acc[pl.ds(a, 16)], m0)
                    acc[pl.ds(a + 16, 16)] = jnp.maximum(acc[pl.ds(a + 16, 16)], m1)
                return 0
            lax.fori_loop(0, Q // 16, grp, 0)
            return (c + 1, go1)

        c_end, _ = lax.while_loop(cond, wbody, (0, go0))
        meta_wait(c_end + 1)
        return 0
    lax.fori_loop(0, NW, frag, 0)

    pltpu.sync_copy(acc.at[pl.ds(0, BUCKET_N * H)],
                    agg_hbm.at[pl.ds(pl.multiple_of(t * BUCKET_N * H, 32), BUCKET_N * H)])


def _sc_segment_max(h, rec_src, rec_dl, rec_w):
    mesh = plsc.VectorSubcoreMesh(core_axis_name="c", subcore_axis_name="s")
    return pl.kernel(
        _scatter_body,
        out_type=jax.ShapeDtypeStruct((N * H,), jnp.float32),
        mesh=mesh,
        compiler_params=pltpu.CompilerParams(needs_layout_passes=False, use_tc_tiling_on_sc=False),
        scratch_types=[
            pltpu.VMEM(((BUCKET_N + 1) * H,), jnp.float32),
            pltpu.VMEM((2, Q, H), jnp.float32),
            pltpu.VMEM((3, Q), jnp.int32),
            pltpu.VMEM((3, Q), jnp.int32),
            pltpu.VMEM((3, Q), jnp.float32),
            pltpu.SemaphoreType.DMA((3,)),
            pltpu.SemaphoreType.DMA((2,)),
        ],
    )(h, rec_src, rec_dl, rec_w)


def kernel(x, edge_index, edge_weight, params):
    p = params
    src = edge_index[0]
    dst = edge_index[1]

    h = _mlp_before(x, p)
    rec_src, rec_dl, rec_w = _sc_partition(src, dst, edge_weight)

    for i in range(NUM_GIN):
        agg = _sc_segment_max(h, rec_src, rec_dl, rec_w).reshape(N, H)

        if i != NUM_GIN - 1:
            h = _gin_dense(h, agg, p["gin_ws"][i], p["gin_bs"][i],
                           p["lm_g"], p["lm_b"], p["gin_eps"][i])
        else:
            h, sums = _gin_dense_final(h, agg, p["gin_ws"][i], p["gin_bs"][i],
                                       p["lm_last_g"], p["lm_last_b"], p["gin_eps"][i])

    pred = _classifier(sums, p)
    return (pred, h)
